# baseline (speedup 1.0000x reference)
import functools

import jax
import jax.numpy as jnp
from jax import lax
from jax.experimental import pallas as pl
from jax.experimental.pallas import tpu as pltpu
from jax.experimental.pallas import tpu_sc as plsc

_N = 10000
_E = 320000
_H = 128
_G = 128
_NL = 3
_NC = 2
_NS = 16
_NW = _NC * _NS
_EPW = _E // _NW
_EPWP = 10240
_PAD = _EPWP - _EPW
_K = 40
_NB = _EPWP // _K
_NCH = 8
_CH = _NB // _NCH
_B = 4
_NP = 10240
_RPT = _NP // _NS


def _leaky(v):
    return jnp.where(v > 0, v, 0.2 * v)


def _bnorm(v, g, b):
    mu = jnp.mean(v, axis=0, keepdims=True)
    var = jnp.mean((v - mu) ** 2, axis=0, keepdims=True)
    return g * (v - mu) / jnp.sqrt(var + 1e-5) + b


def _sc_seg_sum_body(h_hbm, src_hbm, dst_hbm, zero_hbm, out_hbm,
                     srcA, srcB, dstA, dstB, r0, r1, r2, r3, agg_sh,
                     g0, g1, g2, g3, s0, s1, s2, s3, sem_z, sem_is, sem_id):
    cid = lax.axis_index("c")
    sid = lax.axis_index("s")
    wid = sid * _NC + cid
    zslice = agg_sh.at[pl.ds(sid * _RPT, _RPT)]
    srcp = (srcA, srcB)
    dstp = (dstA, dstB)
    rows = (r0, r1, r2, r3)
    gsem = (g0, g1, g2, g3)
    ssem = (s0, s1, s2, s3)

    def gather(iv, l, b):
        pltpu.async_copy(h_hbm.at[iv.at[l]], rows[b], gsem[b])

    def gather_wait(iv, l, b):
        pltpu.make_async_copy(h_hbm.at[iv.at[l]], rows[b], gsem[b]).wait()

    def scat(iv, l, b):
        pltpu.async_copy(rows[b], agg_sh.at[iv.at[l]], ssem[b], add=True)

    def scat_wait(b):
        pltpu.make_async_copy(rows[b], agg_sh.at[dstA.at[0]],
                              ssem[b]).wait()

    def step(srcv, dstv, l, ph, drain=True, nxt=None):
        b = ph % _B
        bp = (ph + 2) % _B
        gather_wait(srcv, l, b)
        scat(dstv, l, b)
        if drain:
            scat_wait(bp)
        if nxt is not None:
            gather(nxt[0], nxt[1], bp)

    pltpu.async_copy(zero_hbm, zslice, sem_z)
    pltpu.sync_copy(src_hbm.at[wid].at[0], srcA)
    pltpu.sync_copy(dst_hbm.at[wid].at[0], dstA)
    gather(srcA, 0, 0)
    gather(srcA, 1, 1)
    pltpu.make_async_copy(zero_hbm, zslice, sem_z).wait()
    plsc.subcore_barrier()

    for ch in range(_NCH):
        q = ch % 2
        srcv, dstv = srcp[q], dstp[q]
        srcn, dstn = srcp[1 - q], dstp[1 - q]
        last = ch + 1 == _NCH

        if ch == 0:
            for l in (0, 1):
                step(srcv, dstv, l, l, drain=False, nxt=(srcv, l + 2))
        else:
            for l in (0, 1):
                step(srcv, dstv, l, l, nxt=(srcv, l + 2))
        if not last:
            pltpu.async_copy(src_hbm.at[wid].at[ch + 1], srcn, sem_is)
            pltpu.async_copy(dst_hbm.at[wid].at[ch + 1], dstn, sem_id)

        def body(t, c, srcv=srcv, dstv=dstv):
            for u in range(_B):
                l = 4 * t + 2 + u
                step(srcv, dstv, l, 2 + u, nxt=(srcv, l + 2))
            return c

        lax.fori_loop(0, (_CH - 8) // _B, body, 0, unroll=False)

        step(srcv, dstv, _CH - 6, _CH - 6, nxt=(srcv, _CH - 4))
        step(srcv, dstv, _CH - 5, _CH - 5, nxt=(srcv, _CH - 3))
        step(srcv, dstv, _CH - 4, _CH - 4, nxt=(srcv, _CH - 2))
        step(srcv, dstv, _CH - 3, _CH - 3, nxt=(srcv, _CH - 1))
        if not last:
            pltpu.make_async_copy(src_hbm.at[wid].at[ch + 1],
                                  srcn, sem_is).wait()
            pltpu.make_async_copy(dst_hbm.at[wid].at[ch + 1],
                                  dstn, sem_id).wait()
            step(srcv, dstv, _CH - 2, _CH - 2, nxt=(srcn, 0))
            step(srcv, dstv, _CH - 1, _CH - 1, nxt=(srcn, 1))
        else:
            step(srcv, dstv, _CH - 2, _CH - 2, drain=False)
            step(srcv, dstv, _CH - 1, _CH - 1, drain=False)
            for b in range(_B):
                scat_wait((_CH - 4 + b) % _B)

    plsc.subcore_barrier()
    pltpu.sync_copy(agg_sh.at[pl.ds(sid * _RPT, _RPT)],
                    out_hbm.at[cid].at[pl.ds(sid * _RPT, _RPT)])


@functools.cache
def _sc_seg_sum():
    return pl.kernel(
        _sc_seg_sum_body,
        out_type=jax.ShapeDtypeStruct((_NC, _NP, _H), jnp.float32),
        mesh=plsc.VectorSubcoreMesh(core_axis_name="c", subcore_axis_name="s"),
        scratch_types=[
            pltpu.VMEM((_CH, _K), jnp.int32),
            pltpu.VMEM((_CH, _K), jnp.int32),
            pltpu.VMEM((_CH, _K), jnp.int32),
            pltpu.VMEM((_CH, _K), jnp.int32),
            pltpu.VMEM((_K, _H), jnp.float32),
            pltpu.VMEM((_K, _H), jnp.float32),
            pltpu.VMEM((_K, _H), jnp.float32),
            pltpu.VMEM((_K, _H), jnp.float32),
            pltpu.VMEM_SHARED((_NP, _H), jnp.float32),
            pltpu.SemaphoreType.DMA,
            pltpu.SemaphoreType.DMA,
            pltpu.SemaphoreType.DMA,
            pltpu.SemaphoreType.DMA,
            pltpu.SemaphoreType.DMA,
            pltpu.SemaphoreType.DMA,
            pltpu.SemaphoreType.DMA,
            pltpu.SemaphoreType.DMA,
            pltpu.SemaphoreType.DMA,
            pltpu.SemaphoreType.DMA,
            pltpu.SemaphoreType.DMA,
        ],
    )


def _mlp_body(h_ref, agg_ref, Wa_ref, ba_ref, g_ref, b_ref, Wb_ref, bb_ref,
              out_ref):
    z = h_ref[...] + agg_ref[0, :_N] + agg_ref[1, :_N]
    z = jnp.dot(z, Wa_ref[...], preferred_element_type=jnp.float32) + ba_ref[...]
    z = _leaky(z)
    z = _bnorm(z, g_ref[...], b_ref[...])
    z = jnp.dot(z, Wb_ref[...], preferred_element_type=jnp.float32) + bb_ref[...]
    out_ref[...] = _leaky(z)


_mlp = pl.pallas_call(
    _mlp_body,
    out_shape=jax.ShapeDtypeStruct((_N, _H), jnp.float32),
)


def _head_body(h_ref, batch_ref, stats_ref, bng_ref, bnb_ref, Wf_ref, bf_ref,
               Wc1_ref, bc1_ref, Wc2_ref, bc2_ref, Wl1a_ref, Wl1b_ref,
               bl1_ref, lg_ref, lb_ref, Wl2_ref, bl2_ref, out_ref):
    seg = batch_ref[...]
    gid = lax.broadcasted_iota(jnp.int32, (_G, _N), 0)
    onehot = jnp.where(seg == gid, 1.0, 0.0)
    pooled = jnp.dot(onehot, h_ref[...], preferred_element_type=jnp.float32)
    o = _bnorm(pooled, bng_ref[...], bnb_ref[...])
    o = jnp.dot(o, Wf_ref[...], preferred_element_type=jnp.float32) + bf_ref[...]
    c = jnp.dot(stats_ref[...], Wc1_ref[...],
                preferred_element_type=jnp.float32) + bc1_ref[...]
    c = jnp.maximum(c, 0.0)
    c = jnp.dot(c, Wc2_ref[...], preferred_element_type=jnp.float32) + bc2_ref[...]
    o2 = (jnp.dot(o, Wl1a_ref[...], preferred_element_type=jnp.float32)
          + jnp.dot(c, Wl1b_ref[...], preferred_element_type=jnp.float32)
          + bl1_ref[...])
    o2 = _leaky(o2)
    o2 = _bnorm(o2, lg_ref[...], lb_ref[...])
    out_ref[...] = jnp.dot(o2, Wl2_ref[...],
                           preferred_element_type=jnp.float32) + bl2_ref[...]


_head = pl.pallas_call(
    _head_body,
    out_shape=jax.ShapeDtypeStruct((_G, _H), jnp.float32),
)


def kernel(x, stats, conv_Wa, conv_ba, conv_g, conv_b, conv_Wb, conv_bb,
           bn_g, bn_b, Wf, bf, Wc1, bc1, Wc2, bc2,
           Wl1, bl1, l_g, l_b, Wl2, bl2, edge_index, batch):
    srcw = edge_index[0].reshape(_NW, _EPW)
    dstw = edge_index[1].reshape(_NW, _EPW)
    wids = jnp.arange(_NW, dtype=jnp.int32)[:, None]
    pad_src = jnp.zeros((_NW, _PAD), jnp.int32)
    pad_dst = _N + 7 * wids + jnp.arange(_PAD, dtype=jnp.int32)[None, :] % 7
    src = jnp.concatenate([srcw, pad_src], 1).reshape(_NW, _NCH, _CH, _K)
    dst = jnp.concatenate([dstw, pad_dst], 1).reshape(_NW, _NCH, _CH, _K)
    zero = jnp.zeros((_RPT, _H), jnp.float32)
    batch2 = batch.reshape(1, _N)

    r = lambda v: v.reshape(1, -1)
    h = x
    for i in range(_NL):
        agg = _sc_seg_sum()(h, src, dst, zero)
        h = _mlp(h, agg, conv_Wa[i], r(conv_ba[i]), r(conv_g[i]),
                 r(conv_b[i]), conv_Wb[i], r(conv_bb[i]))
    return _head(h, batch2, stats, r(bn_g), r(bn_b), Wf, r(bf),
                 Wc1, r(bc1), Wc2, r(bc2), Wl1[:_H], Wl1[_H:], r(bl1),
                 r(l_g), r(l_b), Wl2, r(bl2))


# speedup vs baseline: 2.7363x; 2.7363x over previous
import functools

import jax
import jax.numpy as jnp
from jax import lax
from jax.experimental import pallas as pl
from jax.experimental.pallas import tpu as pltpu
from jax.experimental.pallas import tpu_sc as plsc

_N = 10000
_E = 320000
_H = 128
_G = 128
_NL = 3
_NC = 2
_NS = 16
_NW = _NC * _NS
_EPW = _E // _NW
_K = 40
_NB = _EPW // _K
_NCH = 5
_CH = _NB // _NCH
_B = 4
_NP = 10240
_RPT = _NP // _NS


def _leaky(v):
    return jnp.where(v > 0, v, 0.2 * v)


def _bnorm(v, g, b):
    mu = jnp.mean(v, axis=0, keepdims=True)
    var = jnp.mean((v - mu) ** 2, axis=0, keepdims=True)
    return g * (v - mu) / jnp.sqrt(var + 1e-5) + b


def _sc_seg_sum_body(h_hbm, src_hbm, dst_hbm, zero_hbm, out_hbm,
                     srcA, srcB, dstA, dstB, r0, r1, r2, r3, agg_sh,
                     g0, g1, g2, g3, s0, s1, s2, s3, sem_z, sem_is, sem_id):
    cid = lax.axis_index("c")
    sid = lax.axis_index("s")
    wid = sid * _NC + cid
    zslice = agg_sh.at[pl.ds(sid * _RPT, _RPT)]
    srcp = (srcA, srcB)
    dstp = (dstA, dstB)
    rows = (r0, r1, r2, r3)
    gsem = (g0, g1, g2, g3)
    ssem = (s0, s1, s2, s3)

    def gather(iv, l, b):
        pltpu.async_copy(h_hbm.at[iv.at[l]], rows[b], gsem[b])

    def gather_wait(iv, l, b):
        pltpu.make_async_copy(h_hbm.at[iv.at[l]], rows[b], gsem[b]).wait()

    def scat(iv, l, b):
        pltpu.async_copy(rows[b], agg_sh.at[iv.at[l]], ssem[b], add=True)

    def scat_wait(iv, l, b):
        pltpu.make_async_copy(rows[b], agg_sh.at[iv.at[l]],
                              ssem[b]).wait()

    def step(srcv, dstv, l, ph, dr=None, nxt=None):
        b = ph % _B
        bp = (ph + 2) % _B
        gather_wait(srcv, l, b)
        scat(dstv, l, b)
        if dr is not None:
            scat_wait(dr[0], dr[1], bp)
        if nxt is not None:
            gather(nxt[0], nxt[1], bp)

    pltpu.async_copy(zero_hbm, zslice, sem_z)
    pltpu.sync_copy(src_hbm.at[wid].at[0], srcA)
    pltpu.sync_copy(dst_hbm.at[wid].at[0], dstA)
    gather(srcA, 0, 0)
    gather(srcA, 1, 1)
    pltpu.make_async_copy(zero_hbm, zslice, sem_z).wait()
    plsc.subcore_barrier()

    for ch in range(_NCH):
        q = ch % 2
        srcv, dstv = srcp[q], dstp[q]
        srcn, dstn = srcp[1 - q], dstp[1 - q]
        last = ch + 1 == _NCH
        ph0 = (_CH * ch) % _B

        for l in (0, 1):
            dr = None if ch == 0 else (dstn, _CH - 2 + l)
            step(srcv, dstv, l, ph0 + l, dr=dr, nxt=(srcv, l + 2))
        if not last:
            pltpu.async_copy(src_hbm.at[wid].at[ch + 1], srcn, sem_is)
            pltpu.async_copy(dst_hbm.at[wid].at[ch + 1], dstn, sem_id)

        nf = (_CH - 10) // _B

        def body(t, c, srcv=srcv, dstv=dstv, ph0=ph0):
            for u in range(_B):
                l = 4 * t + 2 + u
                step(srcv, dstv, l, ph0 + 2 + u,
                     dr=(dstv, l - 2), nxt=(srcv, l + 2))
            return c

        lax.fori_loop(0, nf, body, 0, unroll=False)

        for l in range(2 + 4 * nf, _CH - 2):
            step(srcv, dstv, l, ph0 + l, dr=(dstv, l - 2),
                 nxt=(srcv, l + 2))
        if not last:
            pltpu.make_async_copy(src_hbm.at[wid].at[ch + 1],
                                  srcn, sem_is).wait()
            pltpu.make_async_copy(dst_hbm.at[wid].at[ch + 1],
                                  dstn, sem_id).wait()
            step(srcv, dstv, _CH - 2, ph0 + _CH - 2,
                 dr=(dstv, _CH - 4), nxt=(srcn, 0))
            step(srcv, dstv, _CH - 1, ph0 + _CH - 1,
                 dr=(dstv, _CH - 3), nxt=(srcn, 1))
        else:
            step(srcv, dstv, _CH - 2, ph0 + _CH - 2, dr=(dstv, _CH - 4))
            step(srcv, dstv, _CH - 1, ph0 + _CH - 1, dr=(dstv, _CH - 3))
            scat_wait(dstv, _CH - 2, (ph0 + _CH - 2) % _B)
            scat_wait(dstv, _CH - 1, (ph0 + _CH - 1) % _B)

    plsc.subcore_barrier()
    pltpu.sync_copy(agg_sh.at[pl.ds(sid * _RPT, _RPT)],
                    out_hbm.at[cid].at[pl.ds(sid * _RPT, _RPT)])


@functools.cache
def _sc_seg_sum():
    return pl.kernel(
        _sc_seg_sum_body,
        out_type=jax.ShapeDtypeStruct((_NC, _NP, _H), jnp.float32),
        mesh=plsc.VectorSubcoreMesh(core_axis_name="c", subcore_axis_name="s"),
        scratch_types=[
            pltpu.VMEM((_CH, _K), jnp.int32),
            pltpu.VMEM((_CH, _K), jnp.int32),
            pltpu.VMEM((_CH, _K), jnp.int32),
            pltpu.VMEM((_CH, _K), jnp.int32),
            pltpu.VMEM((_K, _H), jnp.float32),
            pltpu.VMEM((_K, _H), jnp.float32),
            pltpu.VMEM((_K, _H), jnp.float32),
            pltpu.VMEM((_K, _H), jnp.float32),
            pltpu.VMEM_SHARED((_NP, _H), jnp.float32),
            pltpu.SemaphoreType.DMA,
            pltpu.SemaphoreType.DMA,
            pltpu.SemaphoreType.DMA,
            pltpu.SemaphoreType.DMA,
            pltpu.SemaphoreType.DMA,
            pltpu.SemaphoreType.DMA,
            pltpu.SemaphoreType.DMA,
            pltpu.SemaphoreType.DMA,
            pltpu.SemaphoreType.DMA,
            pltpu.SemaphoreType.DMA,
            pltpu.SemaphoreType.DMA,
        ],
    )


def _mlp_body(h_ref, agg_ref, Wa_ref, ba_ref, g_ref, b_ref, Wb_ref, bb_ref,
              out_ref):
    z = h_ref[...] + agg_ref[0, :_N] + agg_ref[1, :_N]
    z = jnp.dot(z, Wa_ref[...], preferred_element_type=jnp.float32) + ba_ref[...]
    z = _leaky(z)
    z = _bnorm(z, g_ref[...], b_ref[...])
    z = jnp.dot(z, Wb_ref[...], preferred_element_type=jnp.float32) + bb_ref[...]
    out_ref[...] = _leaky(z)


_mlp = pl.pallas_call(
    _mlp_body,
    out_shape=jax.ShapeDtypeStruct((_N, _H), jnp.float32),
)


def _head_body(h_ref, batch_ref, stats_ref, bng_ref, bnb_ref, Wf_ref, bf_ref,
               Wc1_ref, bc1_ref, Wc2_ref, bc2_ref, Wl1a_ref, Wl1b_ref,
               bl1_ref, lg_ref, lb_ref, Wl2_ref, bl2_ref, out_ref):
    seg = batch_ref[...]
    gid = lax.broadcasted_iota(jnp.int32, (_G, _N), 0)
    onehot = jnp.where(seg == gid, 1.0, 0.0)
    pooled = jnp.dot(onehot, h_ref[...], preferred_element_type=jnp.float32)
    o = _bnorm(pooled, bng_ref[...], bnb_ref[...])
    o = jnp.dot(o, Wf_ref[...], preferred_element_type=jnp.float32) + bf_ref[...]
    c = jnp.dot(stats_ref[...], Wc1_ref[...],
                preferred_element_type=jnp.float32) + bc1_ref[...]
    c = jnp.maximum(c, 0.0)
    c = jnp.dot(c, Wc2_ref[...], preferred_element_type=jnp.float32) + bc2_ref[...]
    o2 = (jnp.dot(o, Wl1a_ref[...], preferred_element_type=jnp.float32)
          + jnp.dot(c, Wl1b_ref[...], preferred_element_type=jnp.float32)
          + bl1_ref[...])
    o2 = _leaky(o2)
    o2 = _bnorm(o2, lg_ref[...], lb_ref[...])
    out_ref[...] = jnp.dot(o2, Wl2_ref[...],
                           preferred_element_type=jnp.float32) + bl2_ref[...]


_head = pl.pallas_call(
    _head_body,
    out_shape=jax.ShapeDtypeStruct((_G, _H), jnp.float32),
)


def kernel(x, stats, conv_Wa, conv_ba, conv_g, conv_b, conv_Wb, conv_bb,
           bn_g, bn_b, Wf, bf, Wc1, bc1, Wc2, bc2,
           Wl1, bl1, l_g, l_b, Wl2, bl2, edge_index, batch):
    src = edge_index[0].reshape(_NW, _NCH, _CH, _K)
    dst = edge_index[1].reshape(_NW, _NCH, _CH, _K)
    zero = jnp.zeros((_RPT, _H), jnp.float32)
    batch2 = batch.reshape(1, _N)

    r = lambda v: v.reshape(1, -1)
    h = x
    for i in range(_NL):
        agg = _sc_seg_sum()(h, src, dst, zero)
        h = _mlp(h, agg, conv_Wa[i], r(conv_ba[i]), r(conv_g[i]),
                 r(conv_b[i]), conv_Wb[i], r(conv_bb[i]))
    return _head(h, batch2, stats, r(bn_g), r(bn_b), Wf, r(bf),
                 Wc1, r(bc1), Wc2, r(bc2), Wl1[:_H], Wl1[_H:], r(bl1),
                 r(l_g), r(l_b), Wl2, r(bl2))


# speedup vs baseline: 3.5862x; 1.3106x over previous
import functools

import jax
import jax.numpy as jnp
from jax import lax
from jax.experimental import pallas as pl
from jax.experimental.pallas import tpu as pltpu
from jax.experimental.pallas import tpu_sc as plsc

_N = 10000
_E = 320000
_H = 128
_G = 128
_NL = 3
_NC = 2
_NS = 16
_NW = _NC * _NS
_EPW = _E // _NW
_K = 80
_NB = _EPW // _K
_NCH = 5
_CH = _NB // _NCH
_B = 3
_NP = 10240
_RPT = _NP // _NS


def _leaky(v):
    return jnp.where(v > 0, v, 0.2 * v)


def _bnorm(v, g, b):
    mu = jnp.mean(v, axis=0, keepdims=True)
    var = jnp.mean((v - mu) ** 2, axis=0, keepdims=True)
    return g * (v - mu) / jnp.sqrt(var + 1e-5) + b


def _sc_seg_sum_body(h_hbm, src_hbm, dst_hbm, zero_hbm, out_hbm,
                     srcA, srcB, dstA, dstB, r0, r1, r2, agg_sh,
                     g0, g1, g2, s0, s1, s2, sem_z, sem_is, sem_id):
    cid = lax.axis_index("c")
    sid = lax.axis_index("s")
    wid = sid * _NC + cid
    zslice = agg_sh.at[pl.ds(sid * _RPT, _RPT)]
    srcp = (srcA, srcB)
    dstp = (dstA, dstB)
    rows = (r0, r1, r2)
    gsem = (g0, g1, g2)
    ssem = (s0, s1, s2)

    def gather(iv, l, b):
        pltpu.async_copy(h_hbm.at[iv.at[l]], rows[b], gsem[b])

    def gather_wait(iv, l, b):
        pltpu.make_async_copy(h_hbm.at[iv.at[l]], rows[b], gsem[b]).wait()

    def scat(iv, l, b):
        pltpu.async_copy(rows[b], agg_sh.at[iv.at[l]], ssem[b], add=True)

    def scat_wait(iv, l, b):
        pltpu.make_async_copy(rows[b], agg_sh.at[iv.at[l]],
                              ssem[b]).wait()

    def step(srcv, dstv, l, ph, dr=None, nxt=None):
        b = ph % _B
        bp = (ph + 2) % _B
        gather_wait(srcv, l, b)
        scat(dstv, l, b)
        if dr is not None:
            scat_wait(dr[0], dr[1], bp)
        if nxt is not None:
            gather(nxt[0], nxt[1], bp)

    pltpu.async_copy(zero_hbm, zslice, sem_z)
    pltpu.sync_copy(src_hbm.at[wid].at[0], srcA)
    pltpu.sync_copy(dst_hbm.at[wid].at[0], dstA)
    gather(srcA, 0, 0)
    gather(srcA, 1, 1)
    pltpu.make_async_copy(zero_hbm, zslice, sem_z).wait()
    plsc.subcore_barrier()

    for ch in range(_NCH):
        q = ch % 2
        srcv, dstv = srcp[q], dstp[q]
        srcn, dstn = srcp[1 - q], dstp[1 - q]
        last = ch + 1 == _NCH
        ph0 = (_CH * ch) % _B

        step(srcv, dstv, 0, ph0,
             dr=None if ch == 0 else (dstn, _CH - 1), nxt=(srcv, 2))
        step(srcv, dstv, 1, ph0 + 1, dr=(dstv, 0), nxt=(srcv, 3))
        if not last:
            pltpu.async_copy(src_hbm.at[wid].at[ch + 1], srcn, sem_is)
            pltpu.async_copy(dst_hbm.at[wid].at[ch + 1], dstn, sem_id)

        nf = (_CH - 4) // _B

        def body(t, c, srcv=srcv, dstv=dstv, ph0=ph0):
            for u in range(_B):
                l = _B * t + 2 + u
                step(srcv, dstv, l, ph0 + 2 + u,
                     dr=(dstv, l - 1), nxt=(srcv, l + 2))
            return c

        lax.fori_loop(0, nf, body, 0, unroll=False)

        for l in range(2 + _B * nf, _CH - 2):
            step(srcv, dstv, l, ph0 + l, dr=(dstv, l - 1),
                 nxt=(srcv, l + 2))
        if not last:
            pltpu.make_async_copy(src_hbm.at[wid].at[ch + 1],
                                  srcn, sem_is).wait()
            pltpu.make_async_copy(dst_hbm.at[wid].at[ch + 1],
                                  dstn, sem_id).wait()
            step(srcv, dstv, _CH - 2, ph0 + _CH - 2,
                 dr=(dstv, _CH - 3), nxt=(srcn, 0))
            step(srcv, dstv, _CH - 1, ph0 + _CH - 1,
                 dr=(dstv, _CH - 2), nxt=(srcn, 1))
        else:
            step(srcv, dstv, _CH - 2, ph0 + _CH - 2, dr=(dstv, _CH - 3))
            step(srcv, dstv, _CH - 1, ph0 + _CH - 1, dr=(dstv, _CH - 2))
            scat_wait(dstv, _CH - 1, (ph0 + _CH - 1) % _B)

    plsc.subcore_barrier()
    pltpu.sync_copy(agg_sh.at[pl.ds(sid * _RPT, _RPT)],
                    out_hbm.at[cid].at[pl.ds(sid * _RPT, _RPT)])


@functools.cache
def _sc_seg_sum():
    return pl.kernel(
        _sc_seg_sum_body,
        out_type=jax.ShapeDtypeStruct((_NC, _NP, _H), jnp.float32),
        mesh=plsc.VectorSubcoreMesh(core_axis_name="c", subcore_axis_name="s"),
        scratch_types=[
            pltpu.VMEM((_CH, _K), jnp.int32),
            pltpu.VMEM((_CH, _K), jnp.int32),
            pltpu.VMEM((_CH, _K), jnp.int32),
            pltpu.VMEM((_CH, _K), jnp.int32),
            pltpu.VMEM((_K, _H), jnp.float32),
            pltpu.VMEM((_K, _H), jnp.float32),
            pltpu.VMEM((_K, _H), jnp.float32),
            pltpu.VMEM_SHARED((_NP, _H), jnp.float32),
            pltpu.SemaphoreType.DMA,
            pltpu.SemaphoreType.DMA,
            pltpu.SemaphoreType.DMA,
            pltpu.SemaphoreType.DMA,
            pltpu.SemaphoreType.DMA,
            pltpu.SemaphoreType.DMA,
            pltpu.SemaphoreType.DMA,
            pltpu.SemaphoreType.DMA,
            pltpu.SemaphoreType.DMA,
        ],
    )


def _mlp_body(h_ref, agg_ref, Wa_ref, ba_ref, g_ref, b_ref, Wb_ref, bb_ref,
              out_ref):
    z = h_ref[...] + agg_ref[0, :_N] + agg_ref[1, :_N]
    z = jnp.dot(z, Wa_ref[...], preferred_element_type=jnp.float32) + ba_ref[...]
    z = _leaky(z)
    z = _bnorm(z, g_ref[...], b_ref[...])
    z = jnp.dot(z, Wb_ref[...], preferred_element_type=jnp.float32) + bb_ref[...]
    out_ref[...] = _leaky(z)


_mlp = pl.pallas_call(
    _mlp_body,
    out_shape=jax.ShapeDtypeStruct((_N, _H), jnp.float32),
)


def _head_body(h_ref, batch_ref, stats_ref, bng_ref, bnb_ref, Wf_ref, bf_ref,
               Wc1_ref, bc1_ref, Wc2_ref, bc2_ref, Wl1a_ref, Wl1b_ref,
               bl1_ref, lg_ref, lb_ref, Wl2_ref, bl2_ref, out_ref):
    seg = batch_ref[...]
    gid = lax.broadcasted_iota(jnp.int32, (_G, _N), 0)
    onehot = jnp.where(seg == gid, 1.0, 0.0)
    pooled = jnp.dot(onehot, h_ref[...], preferred_element_type=jnp.float32)
    o = _bnorm(pooled, bng_ref[...], bnb_ref[...])
    o = jnp.dot(o, Wf_ref[...], preferred_element_type=jnp.float32) + bf_ref[...]
    c = jnp.dot(stats_ref[...], Wc1_ref[...],
                preferred_element_type=jnp.float32) + bc1_ref[...]
    c = jnp.maximum(c, 0.0)
    c = jnp.dot(c, Wc2_ref[...], preferred_element_type=jnp.float32) + bc2_ref[...]
    o2 = (jnp.dot(o, Wl1a_ref[...], preferred_element_type=jnp.float32)
          + jnp.dot(c, Wl1b_ref[...], preferred_element_type=jnp.float32)
          + bl1_ref[...])
    o2 = _leaky(o2)
    o2 = _bnorm(o2, lg_ref[...], lb_ref[...])
    out_ref[...] = jnp.dot(o2, Wl2_ref[...],
                           preferred_element_type=jnp.float32) + bl2_ref[...]


_head = pl.pallas_call(
    _head_body,
    out_shape=jax.ShapeDtypeStruct((_G, _H), jnp.float32),
)


def kernel(x, stats, conv_Wa, conv_ba, conv_g, conv_b, conv_Wb, conv_bb,
           bn_g, bn_b, Wf, bf, Wc1, bc1, Wc2, bc2,
           Wl1, bl1, l_g, l_b, Wl2, bl2, edge_index, batch):
    src = edge_index[0].reshape(_NW, _NCH, _CH, _K)
    dst = edge_index[1].reshape(_NW, _NCH, _CH, _K)
    zero = jnp.zeros((_RPT, _H), jnp.float32)
    batch2 = batch.reshape(1, _N)

    r = lambda v: v.reshape(1, -1)
    h = x
    for i in range(_NL):
        agg = _sc_seg_sum()(h, src, dst, zero)
        h = _mlp(h, agg, conv_Wa[i], r(conv_ba[i]), r(conv_g[i]),
                 r(conv_b[i]), conv_Wb[i], r(conv_bb[i]))
    return _head(h, batch2, stats, r(bn_g), r(bn_b), Wf, r(bf),
                 Wc1, r(bc1), Wc2, r(bc2), Wl1[:_H], Wl1[_H:], r(bl1),
                 r(l_g), r(l_b), Wl2, r(bl2))
